# baseline (speedup 1.0000x reference)
import jax
import jax.numpy as jnp
from jax import lax
from jax.experimental import pallas as pl
from jax.experimental.pallas import tpu as pltpu

N = 4096
D = 512
RB = 1024
NB = N // RB


def _deg_body(spref, m_ref, deg_ref, row_ref):
    i = pl.program_id(0)
    mb = m_ref[...]
    ones = jnp.ones((1, N), dtype=jnp.float32)
    deg_ref[...] = lax.dot_general(
        ones, mb, (((1,), (1,)), ((), ())),
        preferred_element_type=jnp.float32)

    @pl.when(i == 0)
    def _init():
        row_ref[...] = jnp.zeros_like(row_ref)

    rel = spref[0] - i * RB

    @pl.when((rel >= 0) & (rel < RB))
    def _extract_row():
        sel = (lax.broadcasted_iota(jnp.int32, (1, RB), 1)
               == rel).astype(jnp.float32)
        row_ref[...] += jnp.dot(sel, mb, preferred_element_type=jnp.float32)


def _deg_tc(spref, matrix):
    grid_spec = pltpu.PrefetchScalarGridSpec(
        num_scalar_prefetch=1,
        grid=(NB,),
        in_specs=[pl.BlockSpec((RB, N), lambda i, s: (i, 0))],
        out_specs=[
            pl.BlockSpec((1, RB), lambda i, s: (0, i)),
            pl.BlockSpec((1, N), lambda i, s: (0, 0)),
        ],
    )
    return pl.pallas_call(
        _deg_body,
        grid_spec=grid_spec,
        out_shape=[jax.ShapeDtypeStruct((1, N), jnp.float32),
                   jax.ShapeDtypeStruct((1, N), jnp.float32)],
    )(spref, matrix)


def _main_body(spref, deg_ref, row_ref, x_ref, lmT_ref,
               W1_ref, b1_ref, W2_ref, b2_ref, Wp_ref, bp_ref, out_ref):
    row = row_ref[...]
    nb = row != 0
    wt = jnp.where(nb, lax.rsqrt(jnp.where(nb, deg_ref[...], 1.0)), 0.0)
    lwT = lmT_ref[...] * wt
    xw = x_ref[...]
    A = jnp.dot(lwT, xw, preferred_element_type=jnp.float32)
    sc = jnp.sum(lwT, axis=1, keepdims=True)
    S = jnp.sum(row)
    rs = jnp.where(S > 0, lax.rsqrt(S), 0.0)
    idx = spref[0]
    sel = (lax.broadcasted_iota(jnp.int32, (1, N), 1)
           == idx).astype(jnp.float32)
    XI = jnp.dot(sel, xw, preferred_element_type=jnp.float32)
    flagv = spref[1]
    Wsel = jnp.where(flagv == 1, W1_ref[...], W2_ref[...])
    bsel = jnp.where(flagv == 1, b1_ref[...], b2_ref[...])
    SB = (sc * rs) * bsel
    ta = jnp.maximum(
        jnp.dot(A * rs, Wsel, preferred_element_type=jnp.float32) + SB, 0.0)
    zi = jnp.maximum(
        jnp.dot(XI, Wsel, preferred_element_type=jnp.float32) + bsel, 0.0)
    h = jnp.concatenate(
        [zi] + [ta[l:l + 1, :] for l in range(7)], axis=1)
    P = jnp.maximum(XI, h)
    out_ref[...] = (jnp.dot(P, Wp_ref[...],
                            preferred_element_type=jnp.float32)
                    + bp_ref[...])


def _main_tc(spref, deg_row, mrow, x, lmT8, W1, b1, W2, b2, Wp, bp):
    grid_spec = pltpu.PrefetchScalarGridSpec(
        num_scalar_prefetch=1,
        grid=(1,),
        in_specs=[
            pl.BlockSpec((1, N), lambda i, s: (0, 0)),
            pl.BlockSpec((1, N), lambda i, s: (0, 0)),
            pl.BlockSpec((N, D), lambda i, s: (0, 0)),
            pl.BlockSpec((8, N), lambda i, s: (0, 0)),
            pl.BlockSpec((D, 64), lambda i, s: (0, 0)),
            pl.BlockSpec((1, 64), lambda i, s: (0, 0)),
            pl.BlockSpec((D, 64), lambda i, s: (0, 0)),
            pl.BlockSpec((1, 64), lambda i, s: (0, 0)),
            pl.BlockSpec((D, 7), lambda i, s: (0, 0)),
            pl.BlockSpec((1, 7), lambda i, s: (0, 0)),
        ],
        out_specs=pl.BlockSpec((1, 7), lambda i, s: (0, 0)),
    )
    return pl.pallas_call(
        _main_body,
        grid_spec=grid_spec,
        out_shape=jax.ShapeDtypeStruct((1, 7), jnp.float32),
    )(spref, deg_row, mrow, x, lmT8, W1, b1, W2, b2, Wp, bp)


def kernel(flag, index, matrix, x_features, x_labels, W1, b1, W2, b2, Wp, bp):
    spref = jnp.array([index, flag]).astype(jnp.int32)
    deg_row, mrow = _deg_tc(spref, matrix)
    lmT = (x_labels != 0).astype(jnp.float32).T
    lmT8 = jnp.concatenate(
        [lmT, jnp.zeros((1, N), jnp.float32)], axis=0)
    return _main_tc(spref, deg_row, mrow, x_features, lmT8,
                    W1, b1.reshape(1, 64), W2, b2.reshape(1, 64),
                    Wp, bp.reshape(1, 7))


# speedup vs baseline: 1.0298x; 1.0298x over previous
import jax
import jax.numpy as jnp
from jax import lax
from jax.experimental import pallas as pl
from jax.experimental.pallas import tpu as pltpu

N = 4096
D = 512
RB = 512
NB = N // RB


def _deg_body(spref, m_ref, deg_ref, row_ref):
    i = pl.program_id(0)
    mb = m_ref[...]
    ones = jnp.ones((1, N), dtype=jnp.float32)
    deg_ref[...] = lax.dot_general(
        ones, mb, (((1,), (1,)), ((), ())),
        preferred_element_type=jnp.float32)

    @pl.when(i == 0)
    def _init():
        row_ref[...] = jnp.zeros_like(row_ref)

    rel = spref[0] - i * RB

    @pl.when((rel >= 0) & (rel < RB))
    def _extract_row():
        sel = (lax.broadcasted_iota(jnp.int32, (1, RB), 1)
               == rel).astype(jnp.float32)
        row_ref[...] += jnp.dot(sel, mb, preferred_element_type=jnp.float32)


def _deg_tc(spref, matrix):
    grid_spec = pltpu.PrefetchScalarGridSpec(
        num_scalar_prefetch=1,
        grid=(NB,),
        in_specs=[pl.BlockSpec((RB, N), lambda i, s: (i, 0))],
        out_specs=[
            pl.BlockSpec((1, RB), lambda i, s: (0, i)),
            pl.BlockSpec((1, N), lambda i, s: (0, 0)),
        ],
    )
    return pl.pallas_call(
        _deg_body,
        grid_spec=grid_spec,
        out_shape=[jax.ShapeDtypeStruct((1, N), jnp.float32),
                   jax.ShapeDtypeStruct((1, N), jnp.float32)],
    )(spref, matrix)


def _main_body(spref, deg_ref, row_ref, x_ref, lmT_ref,
               W1_ref, b1_ref, W2_ref, b2_ref, Wp_ref, bp_ref, out_ref):
    row = row_ref[...]
    nb = row != 0
    wt = jnp.where(nb, lax.rsqrt(jnp.where(nb, deg_ref[...], 1.0)), 0.0)
    lwT = lmT_ref[...] * wt
    xw = x_ref[...]
    A = jnp.dot(lwT, xw, preferred_element_type=jnp.float32)
    sc = jnp.sum(lwT, axis=1, keepdims=True)
    S = jnp.sum(row)
    rs = jnp.where(S > 0, lax.rsqrt(S), 0.0)
    idx = spref[0]
    sel = (lax.broadcasted_iota(jnp.int32, (1, N), 1)
           == idx).astype(jnp.float32)
    XI = jnp.dot(sel, xw, preferred_element_type=jnp.float32)
    flagv = spref[1]
    Wsel = jnp.where(flagv == 1, W1_ref[...], W2_ref[...])
    bsel = jnp.where(flagv == 1, b1_ref[...], b2_ref[...])
    SB = (sc * rs) * bsel
    ta = jnp.maximum(
        jnp.dot(A * rs, Wsel, preferred_element_type=jnp.float32) + SB, 0.0)
    zi = jnp.maximum(
        jnp.dot(XI, Wsel, preferred_element_type=jnp.float32) + bsel, 0.0)
    h = jnp.concatenate(
        [zi] + [ta[l:l + 1, :] for l in range(7)], axis=1)
    P = jnp.maximum(XI, h)
    out_ref[...] = (jnp.dot(P, Wp_ref[...],
                            preferred_element_type=jnp.float32)
                    + bp_ref[...])


def _main_tc(spref, deg_row, mrow, x, lmT8, W1, b1, W2, b2, Wp, bp):
    grid_spec = pltpu.PrefetchScalarGridSpec(
        num_scalar_prefetch=1,
        grid=(1,),
        in_specs=[
            pl.BlockSpec((1, N), lambda i, s: (0, 0)),
            pl.BlockSpec((1, N), lambda i, s: (0, 0)),
            pl.BlockSpec((N, D), lambda i, s: (0, 0)),
            pl.BlockSpec((8, N), lambda i, s: (0, 0)),
            pl.BlockSpec((D, 64), lambda i, s: (0, 0)),
            pl.BlockSpec((1, 64), lambda i, s: (0, 0)),
            pl.BlockSpec((D, 64), lambda i, s: (0, 0)),
            pl.BlockSpec((1, 64), lambda i, s: (0, 0)),
            pl.BlockSpec((D, 7), lambda i, s: (0, 0)),
            pl.BlockSpec((1, 7), lambda i, s: (0, 0)),
        ],
        out_specs=pl.BlockSpec((1, 7), lambda i, s: (0, 0)),
    )
    return pl.pallas_call(
        _main_body,
        grid_spec=grid_spec,
        out_shape=jax.ShapeDtypeStruct((1, 7), jnp.float32),
    )(spref, deg_row, mrow, x, lmT8, W1, b1, W2, b2, Wp, bp)


def kernel(flag, index, matrix, x_features, x_labels, W1, b1, W2, b2, Wp, bp):
    spref = jnp.array([index, flag]).astype(jnp.int32)
    deg_row, mrow = _deg_tc(spref, matrix)
    lmT = (x_labels != 0).astype(jnp.float32).T
    lmT8 = jnp.concatenate(
        [lmT, jnp.zeros((1, N), jnp.float32)], axis=0)
    return _main_tc(spref, deg_row, mrow, x_features, lmT8,
                    W1, b1.reshape(1, 64), W2, b2.reshape(1, 64),
                    Wp, bp.reshape(1, 7))
